# baseline (speedup 1.0000x reference)
import jax
import jax.numpy as jnp
from jax import lax
from jax.experimental import pallas as pl
from jax.experimental.pallas import tpu as pltpu
from jax.experimental.pallas import tpu_sc as plsc

N = 10000
E = 320000
D = 128
C = 10
G = 64

NC = 2
NS = 16
NW = NC * NS
K = 64
CH = 160
EPW = K * CH
EPAD = NW * EPW
NBUF = 4
RPS = 624
RTAIL = N - RPS * NS
LANES = 16


def _sc_body(x_hbm, sdw_hbm, zero_hbm, out_hbm,
             sdw0, sdw1, sdw2, sdw3, rows0, rows1, rows2, rows3, wtmp_v,
             agg_sh, gs0, gs1, gs2, gs3, ss0, ss1, ss2, ss3):
    c = lax.axis_index("c")
    s = lax.axis_index("s")
    wid = c * NS + s

    sdw = (sdw0, sdw1, sdw2, sdw3)
    rows = (rows0, rows1, rows2, rows3)
    gsem = (gs0, gs1, gs2, gs3)
    ssem = (ss0, ss1, ss2, ss3)

    pltpu.sync_copy(zero_hbm.at[pl.ds(s * RPS, RPS)],
                    agg_sh.at[pl.ds(s * RPS, RPS)])

    @pl.when(s == NS - 1)
    def _zero_tail():
        pltpu.sync_copy(zero_hbm.at[pl.ds(RPS * NS, RTAIL)],
                        agg_sh.at[pl.ds(RPS * NS, RTAIL)])

    plsc.subcore_barrier()

    def launch(g, b):
        pltpu.sync_copy(sdw_hbm.at[wid, g], sdw[b])
        pltpu.make_async_copy(x_hbm.at[sdw[b].at[0]], rows[b], gsem[b]).start()

    def retire(g, b):
        pltpu.make_async_copy(x_hbm.at[sdw[b].at[0]], rows[b], gsem[b]).wait()
        for q in range(K // LANES):
            wv = sdw[b][2, pl.ds(q * LANES, LANES)]
            wtmp_v[pl.ds(8 + q * LANES, LANES)] = plsc.bitcast(wv, jnp.float32)

        @plsc.parallel_loop(0, K, unroll=4)
        def _scale(e):
            eidx = jnp.full((LANES,), 0, dtype=jnp.int32) + (e + 8)
            wspl = plsc.load_gather(wtmp_v, [eidx])
            for j in range(D // LANES):
                sl = rows[b][e, pl.ds(j * LANES, LANES)]
                rows[b][e, pl.ds(j * LANES, LANES)] = sl * wspl

        pltpu.async_copy(rows[b], agg_sh.at[sdw[b].at[1]], ssem[b],
                         add=True)

    def wait_scatter(b):
        pltpu.make_async_copy(rows[b], agg_sh.at[sdw[b].at[1]],
                              ssem[b]).wait()

    launch(0, 0)
    launch(1, 1)
    launch(2, 2)

    retire(0, 0)
    launch(3, 3)
    retire(1, 1)
    wait_scatter(0)
    launch(4, 0)
    retire(2, 2)
    wait_scatter(1)
    launch(5, 1)
    retire(3, 3)
    wait_scatter(2)
    launch(6, 2)

    def quad_body(qq, carry):
        g0 = qq * 4 + 4
        retire(g0 + 0, 0)
        wait_scatter(3)
        launch(g0 + 3, 3)
        retire(g0 + 1, 1)
        wait_scatter(0)
        launch(g0 + 4, 0)
        retire(g0 + 2, 2)
        wait_scatter(1)
        launch(g0 + 5, 1)
        retire(g0 + 3, 3)
        wait_scatter(2)
        launch(g0 + 6, 2)
        return carry

    lax.fori_loop(0, CH // 4 - 2, quad_body, 0)
    g0 = CH - 4
    retire(g0 + 0, 0)
    wait_scatter(3)
    launch(g0 + 3, 3)
    retire(g0 + 1, 1)
    retire(g0 + 2, 2)
    retire(g0 + 3, 3)
    wait_scatter(0)
    wait_scatter(1)
    wait_scatter(2)
    wait_scatter(3)
    plsc.subcore_barrier()

    pltpu.sync_copy(agg_sh.at[pl.ds(s * RPS, RPS)],
                    out_hbm.at[c, pl.ds(s * RPS, RPS)])

    @pl.when(s == NS - 1)
    def _flush_tail():
        pltpu.sync_copy(agg_sh.at[pl.ds(RPS * NS, RTAIL)],
                        out_hbm.at[c, pl.ds(RPS * NS, RTAIL)])


_sc_aggregate = pl.kernel(
    _sc_body,
    out_type=jax.ShapeDtypeStruct((NC, N, D), jnp.float32),
    mesh=plsc.VectorSubcoreMesh(
        core_axis_name="c", subcore_axis_name="s",
        num_cores=NC, num_subcores=NS),
    scratch_types=[
        pltpu.VMEM((3, K), jnp.int32),
        pltpu.VMEM((3, K), jnp.int32),
        pltpu.VMEM((3, K), jnp.int32),
        pltpu.VMEM((3, K), jnp.int32),
        pltpu.VMEM((K, D), jnp.float32),
        pltpu.VMEM((K, D), jnp.float32),
        pltpu.VMEM((K, D), jnp.float32),
        pltpu.VMEM((K, D), jnp.float32),
        pltpu.VMEM((K + 8,), jnp.float32),
        pltpu.VMEM_SHARED((N, D), jnp.float32),
        pltpu.SemaphoreType.DMA,
        pltpu.SemaphoreType.DMA,
        pltpu.SemaphoreType.DMA,
        pltpu.SemaphoreType.DMA,
        pltpu.SemaphoreType.DMA,
        pltpu.SemaphoreType.DMA,
        pltpu.SemaphoreType.DMA,
        pltpu.SemaphoreType.DMA,
    ],
    compiler_params=pltpu.CompilerParams(needs_layout_passes=False),
)


def _tc_head_body(agg_ref, batch_ref, wenc_ref, benc_ref, wcls_ref, bcls_ref,
                  out_ref):
    agg = agg_ref[0] + agg_ref[1]
    h = jnp.dot(agg, wenc_ref[...], preferred_element_type=jnp.float32)
    h = jnp.maximum(h + benc_ref[...], 0.0)
    bt = batch_ref[...]
    gids = lax.broadcasted_iota(jnp.int32, (G, N), 0)
    oh = (gids == bt).astype(jnp.float32)
    pooled_sum = jnp.dot(oh, h, preferred_element_type=jnp.float32)
    counts = jnp.sum(oh, axis=1, keepdims=True)
    pooled = pooled_sum / jnp.maximum(counts, 1.0)
    logits = jnp.dot(pooled, wcls_ref[...],
                     preferred_element_type=jnp.float32) + bcls_ref[...]
    out_ref[...] = logits


_tc_head = pl.pallas_call(
    _tc_head_body,
    out_shape=jax.ShapeDtypeStruct((G, C), jnp.float32),
)


def kernel(x, edge_index, batch, edge_weight, W_enc, b_enc, W_cls, b_cls):
    pad = EPAD - E
    src = jnp.pad(edge_index[0], (0, pad)).reshape(NW, CH, K)
    dst = jnp.pad(edge_index[1], (0, pad)).reshape(NW, CH, K)
    wbits = lax.bitcast_convert_type(jnp.pad(edge_weight, (0, pad)),
                                     jnp.int32).reshape(NW, CH, K)
    sdw = jnp.stack([src, dst, wbits], axis=2)
    zeros_nd = jnp.zeros((N, D), jnp.float32)
    agg2 = _sc_aggregate(x, sdw, zeros_nd)
    return _tc_head(agg2, batch.reshape(1, N), W_enc, b_enc.reshape(1, D),
                    W_cls, b_cls.reshape(1, C))


# speedup vs baseline: 1.5285x; 1.5285x over previous
import functools

import jax
import jax.numpy as jnp
from jax import lax
from jax.experimental import pallas as pl
from jax.experimental.pallas import tpu as pltpu
from jax.experimental.pallas import tpu_sc as plsc

N = 10000
E = 320000
D = 128
C = 10
G = 64

NC = 2
NS = 16
NW = NC * NS
K = 128
CHUNKS = 79
EPW = K * CHUNKS
EPAD = NW * EPW
RPS = 624
RTAIL = N - RPS * NS
LANES = 16


def _sc_body(x_hbm, sd_hbm, zero_hbm, out_hbm,
             sd0_v, sd1_v, wtmp_v, rows0_v, rows1_v, agg_sh, sem0, sem1):
    c = lax.axis_index("c")
    s = lax.axis_index("s")
    wid = c * NS + s

    pltpu.sync_copy(zero_hbm.at[pl.ds(s * RPS, RPS)],
                    agg_sh.at[pl.ds(s * RPS, RPS)])

    @pl.when(s == NS - 1)
    def _zero_tail():
        pltpu.sync_copy(zero_hbm.at[pl.ds(RPS * NS, RTAIL)],
                        agg_sh.at[pl.ds(RPS * NS, RTAIL)])

    plsc.subcore_barrier()

    def load_sd(g, sd_v):
        pltpu.sync_copy(sd_hbm.at[wid, g], sd_v)

    def gather(sd_v, rows_v, sem):
        return pltpu.make_async_copy(x_hbm.at[sd_v.at[0]], rows_v, sem)

    def scale_and_scatter(g, rows_v, sd_v):
        for q in range(K // LANES):
            wv = sd_v[2, pl.ds(q * LANES, LANES)]
            wtmp_v[pl.ds(8 + q * LANES, LANES)] = plsc.bitcast(wv, jnp.float32)

        @plsc.parallel_loop(0, K, unroll=4)
        def _scale(e):
            eidx = jnp.full((LANES,), 0, dtype=jnp.int32) + (e + 8)
            wspl = plsc.load_gather(wtmp_v, [eidx])
            for j in range(D // LANES):
                sl = rows_v[e, pl.ds(j * LANES, LANES)]
                rows_v[e, pl.ds(j * LANES, LANES)] = sl * wspl

        pltpu.sync_copy(rows_v, agg_sh.at[sd_v.at[1]], add=True)

    load_sd(0, sd0_v)
    gather(sd0_v, rows0_v, sem0).start()

    def pair_body(h, carry):
        g0 = h * 2
        load_sd(g0 + 1, sd1_v)
        gather(sd1_v, rows1_v, sem1).start()
        gather(sd0_v, rows0_v, sem0).wait()
        scale_and_scatter(g0, rows0_v, sd0_v)
        load_sd(g0 + 2, sd0_v)
        gather(sd0_v, rows0_v, sem0).start()
        gather(sd1_v, rows1_v, sem1).wait()
        scale_and_scatter(g0 + 1, rows1_v, sd1_v)
        return carry

    lax.fori_loop(0, CHUNKS // 2, pair_body, 0)
    gather(sd0_v, rows0_v, sem0).wait()
    scale_and_scatter(CHUNKS - 1, rows0_v, sd0_v)
    plsc.subcore_barrier()

    pltpu.sync_copy(agg_sh.at[pl.ds(s * RPS, RPS)],
                    out_hbm.at[c, pl.ds(s * RPS, RPS)])

    @pl.when(s == NS - 1)
    def _flush_tail():
        pltpu.sync_copy(agg_sh.at[pl.ds(RPS * NS, RTAIL)],
                        out_hbm.at[c, pl.ds(RPS * NS, RTAIL)])


_sc_aggregate = pl.kernel(
    _sc_body,
    out_type=jax.ShapeDtypeStruct((NC, N, D), jnp.float32),
    mesh=plsc.VectorSubcoreMesh(
        core_axis_name="c", subcore_axis_name="s",
        num_cores=NC, num_subcores=NS),
    scratch_types=[
        pltpu.VMEM((3, K), jnp.int32),
        pltpu.VMEM((3, K), jnp.int32),
        pltpu.VMEM((K + 8,), jnp.float32),
        pltpu.VMEM((K, D), jnp.float32),
        pltpu.VMEM((K, D), jnp.float32),
        pltpu.VMEM_SHARED((N, D), jnp.float32),
        pltpu.SemaphoreType.DMA,
        pltpu.SemaphoreType.DMA,
    ],
    compiler_params=pltpu.CompilerParams(needs_layout_passes=False),
)


def _tc_head_body(agg_ref, batch_ref, wenc_ref, benc_ref, wcls_ref, bcls_ref,
                  out_ref):
    agg = agg_ref[0] + agg_ref[1]
    h = jnp.dot(agg, wenc_ref[...], preferred_element_type=jnp.float32)
    h = jnp.maximum(h + benc_ref[...], 0.0)
    bt = batch_ref[...]
    gids = lax.broadcasted_iota(jnp.int32, (G, N), 0)
    oh = (gids == bt).astype(jnp.float32)
    pooled_sum = jnp.dot(oh, h, preferred_element_type=jnp.float32)
    counts = jnp.sum(oh, axis=1, keepdims=True)
    pooled = pooled_sum / jnp.maximum(counts, 1.0)
    logits = jnp.dot(pooled, wcls_ref[...],
                     preferred_element_type=jnp.float32) + bcls_ref[...]
    out_ref[...] = logits


_tc_head = pl.pallas_call(
    _tc_head_body,
    out_shape=jax.ShapeDtypeStruct((G, C), jnp.float32),
)


def kernel(x, edge_index, batch, edge_weight, W_enc, b_enc, W_cls, b_cls):
    pad = EPAD - E
    src = jnp.pad(edge_index[0], (0, pad)).reshape(NW, CHUNKS, K)
    dst = jnp.pad(edge_index[1], (0, pad)).reshape(NW, CHUNKS, K)
    wbits = lax.bitcast_convert_type(jnp.pad(edge_weight, (0, pad)),
                                     jnp.int32).reshape(NW, CHUNKS, K)
    sd = jnp.stack([src, dst, wbits], axis=2)
    zeros_nd = jnp.zeros((N, D), jnp.float32)
    agg2 = _sc_aggregate(x, sd, zeros_nd)
    return _tc_head(agg2, batch.reshape(1, N), W_enc, b_enc.reshape(1, D),
                    W_cls, b_cls.reshape(1, C))


# speedup vs baseline: 1.8561x; 1.2143x over previous
import jax
import jax.numpy as jnp
from jax import lax
from jax.experimental import pallas as pl
from jax.experimental.pallas import tpu as pltpu
from jax.experimental.pallas import tpu_sc as plsc

N = 10000
E = 320000
D = 128
C = 10
G = 64

NC = 2
NS = 16
NW = NC * NS
K = 64
CHUNKS = 157
EPW = K * CHUNKS
EPAD = NW * EPW
RPS = 624
RTAIL = N - RPS * NS
LANES = 16


def _sc_body(x_hbm, sd_hbm, zero_hbm, out_hbm,
             sd0_v, sd1_v, wtmp_v, rows0_v, rows1_v, fb0_v, fb1_v,
             di0_v, di1_v, agg_sh, gs0, gs1, fs0, fs1):
    c = lax.axis_index("c")
    s = lax.axis_index("s")
    wid = c * NS + s

    sd = (sd0_v, sd1_v)
    rowsb = (rows0_v, rows1_v)
    fb = (fb0_v, fb1_v)
    di = (di0_v, di1_v)
    gsem = (gs0, gs1)
    fsem = (fs0, fs1)

    pltpu.sync_copy(zero_hbm.at[pl.ds(s * RPS, RPS)],
                    agg_sh.at[pl.ds(s * RPS, RPS)])

    @pl.when(s == NS - 1)
    def _zero_tail():
        pltpu.sync_copy(zero_hbm.at[pl.ds(RPS * NS, RTAIL)],
                        agg_sh.at[pl.ds(RPS * NS, RTAIL)])

    plsc.subcore_barrier()

    def load_sd(g, p):
        pltpu.sync_copy(sd_hbm.at[wid, g], sd[p])

    def start_gather(p):
        pltpu.make_async_copy(x_hbm.at[sd[p].at[0]], rowsb[p],
                              gsem[p]).start()

    def process(p, first):
        pltpu.make_async_copy(x_hbm.at[sd[p].at[0]], rowsb[p],
                              gsem[p]).wait()
        if not first:
            pltpu.make_async_copy(fb[p], agg_sh.at[di[p]], fsem[p]).wait()
        for q in range(K // LANES):
            wv = sd[p][2, pl.ds(q * LANES, LANES)]
            wtmp_v[pl.ds(8 + q * LANES, LANES)] = plsc.bitcast(wv, jnp.float32)
            di[p][pl.ds(q * LANES, LANES)] = sd[p][1, pl.ds(q * LANES, LANES)]

        @plsc.parallel_loop(0, K, unroll=4)
        def _scale(e):
            eidx = jnp.full((LANES,), 0, dtype=jnp.int32) + (e + 8)
            wspl = plsc.load_gather(wtmp_v, [eidx])
            for j in range(D // LANES):
                sl = rowsb[p][e, pl.ds(j * LANES, LANES)]
                fb[p][e, pl.ds(j * LANES, LANES)] = sl * wspl

        pltpu.async_copy(fb[p], agg_sh.at[di[p]], fsem[p], add=True)

    load_sd(0, 0)
    start_gather(0)
    load_sd(1, 1)
    start_gather(1)
    process(0, True)
    load_sd(2, 0)
    start_gather(0)
    process(1, True)

    def pair_body(h, carry):
        g0 = h * 2 + 2
        load_sd(g0 + 1, 1)
        start_gather(1)
        process(0, False)
        load_sd(g0 + 2, 0)
        start_gather(0)
        process(1, False)
        return carry

    lax.fori_loop(0, CHUNKS // 2 - 1, pair_body, 0)
    process(0, False)
    pltpu.make_async_copy(fb[0], agg_sh.at[di[0]], fsem[0]).wait()
    pltpu.make_async_copy(fb[1], agg_sh.at[di[1]], fsem[1]).wait()
    plsc.subcore_barrier()

    pltpu.sync_copy(agg_sh.at[pl.ds(s * RPS, RPS)],
                    out_hbm.at[c, pl.ds(s * RPS, RPS)])

    @pl.when(s == NS - 1)
    def _flush_tail():
        pltpu.sync_copy(agg_sh.at[pl.ds(RPS * NS, RTAIL)],
                        out_hbm.at[c, pl.ds(RPS * NS, RTAIL)])


_sc_aggregate = pl.kernel(
    _sc_body,
    out_type=jax.ShapeDtypeStruct((NC, N, D), jnp.float32),
    mesh=plsc.VectorSubcoreMesh(
        core_axis_name="c", subcore_axis_name="s",
        num_cores=NC, num_subcores=NS),
    scratch_types=[
        pltpu.VMEM((3, K), jnp.int32),
        pltpu.VMEM((3, K), jnp.int32),
        pltpu.VMEM((K + 8,), jnp.float32),
        pltpu.VMEM((K, D), jnp.float32),
        pltpu.VMEM((K, D), jnp.float32),
        pltpu.VMEM((K, D), jnp.float32),
        pltpu.VMEM((K, D), jnp.float32),
        pltpu.VMEM((K,), jnp.int32),
        pltpu.VMEM((K,), jnp.int32),
        pltpu.VMEM_SHARED((N, D), jnp.float32),
        pltpu.SemaphoreType.DMA,
        pltpu.SemaphoreType.DMA,
        pltpu.SemaphoreType.DMA,
        pltpu.SemaphoreType.DMA,
    ],
    compiler_params=pltpu.CompilerParams(needs_layout_passes=False),
)


def _tc_head_body(agg_ref, batch_ref, wenc_ref, benc_ref, wcls_ref, bcls_ref,
                  out_ref):
    agg = agg_ref[0] + agg_ref[1]
    h = jnp.dot(agg, wenc_ref[...], preferred_element_type=jnp.float32)
    h = jnp.maximum(h + benc_ref[...], 0.0)
    bt = batch_ref[...]
    gids = lax.broadcasted_iota(jnp.int32, (G, N), 0)
    oh = (gids == bt).astype(jnp.float32)
    pooled_sum = jnp.dot(oh, h, preferred_element_type=jnp.float32)
    counts = jnp.sum(oh, axis=1, keepdims=True)
    pooled = pooled_sum / jnp.maximum(counts, 1.0)
    logits = jnp.dot(pooled, wcls_ref[...],
                     preferred_element_type=jnp.float32) + bcls_ref[...]
    out_ref[...] = logits


_tc_head = pl.pallas_call(
    _tc_head_body,
    out_shape=jax.ShapeDtypeStruct((G, C), jnp.float32),
)


def kernel(x, edge_index, batch, edge_weight, W_enc, b_enc, W_cls, b_cls):
    pad = EPAD - E
    src = jnp.pad(edge_index[0], (0, pad)).reshape(NW, CHUNKS, K)
    dst = jnp.pad(edge_index[1], (0, pad)).reshape(NW, CHUNKS, K)
    wbits = lax.bitcast_convert_type(jnp.pad(edge_weight, (0, pad)),
                                     jnp.int32).reshape(NW, CHUNKS, K)
    sd = jnp.stack([src, dst, wbits], axis=2)
    zeros_nd = jnp.zeros((N, D), jnp.float32)
    agg2 = _sc_aggregate(x, sd, zeros_nd)
    return _tc_head(agg2, batch.reshape(1, N), W_enc, b_enc.reshape(1, D),
                    W_cls, b_cls.reshape(1, C))


# speedup vs baseline: 2.2199x; 1.1960x over previous
import functools

import jax
import jax.numpy as jnp
from jax import lax
from jax.experimental import pallas as pl
from jax.experimental.pallas import tpu as pltpu
from jax.experimental.pallas import tpu_sc as plsc

N = 10000
E = 320000
D = 128
C = 10
G = 64

NC = 2
NS = 16
NW = NC * NS
EPW = E // NW
K = 80
CHUNKS = EPW // K
RPS = 624
RTAIL = N - RPS * NS
LANES = 16


def _sc_body(x_hbm, sd_hbm, w_hbm, zero_hbm, out_hbm,
             sd0_v, sd1_v, w_v, rows0_v, rows1_v, agg_sh, sem0, sem1):
    c = lax.axis_index("c")
    s = lax.axis_index("s")
    wid = c * NS + s

    pltpu.sync_copy(w_hbm.at[pl.ds(wid * EPW, EPW)], w_v.at[pl.ds(8, EPW)])

    pltpu.sync_copy(zero_hbm.at[pl.ds(s * RPS, RPS)],
                    agg_sh.at[pl.ds(s * RPS, RPS)])

    @pl.when(s == NS - 1)
    def _zero_tail():
        pltpu.sync_copy(zero_hbm.at[pl.ds(RPS * NS, RTAIL)],
                        agg_sh.at[pl.ds(RPS * NS, RTAIL)])

    plsc.subcore_barrier()

    def load_sd(g, sd_v):
        pltpu.sync_copy(sd_hbm.at[wid, g], sd_v)

    def gather(sd_v, rows_v, sem):
        return pltpu.make_async_copy(x_hbm.at[sd_v.at[0]], rows_v, sem)

    def scale_and_scatter(g, rows_v, sd_v):
        wbase = g * K + 8

        @plsc.parallel_loop(0, K, unroll=4)
        def _scale(e):
            eidx = jnp.full((LANES,), 0, dtype=jnp.int32) + (e + wbase)
            wspl = plsc.load_gather(w_v, [eidx])
            for j in range(D // LANES):
                sl = rows_v[e, pl.ds(j * LANES, LANES)]
                rows_v[e, pl.ds(j * LANES, LANES)] = sl * wspl

        pltpu.sync_copy(rows_v, agg_sh.at[sd_v.at[1]], add=True)

    load_sd(0, sd0_v)
    gather(sd0_v, rows0_v, sem0).start()

    def pair_body(h, carry):
        g0 = h * 2
        load_sd(g0 + 1, sd1_v)
        gather(sd1_v, rows1_v, sem1).start()
        gather(sd0_v, rows0_v, sem0).wait()
        scale_and_scatter(g0, rows0_v, sd0_v)
        load_sd(g0 + 2, sd0_v)
        gather(sd0_v, rows0_v, sem0).start()
        gather(sd1_v, rows1_v, sem1).wait()
        scale_and_scatter(g0 + 1, rows1_v, sd1_v)
        return carry

    lax.fori_loop(0, CHUNKS // 2, pair_body, 0)
    gather(sd0_v, rows0_v, sem0).wait()
    scale_and_scatter(CHUNKS - 1, rows0_v, sd0_v)
    plsc.subcore_barrier()

    pltpu.sync_copy(agg_sh.at[pl.ds(s * RPS, RPS)],
                    out_hbm.at[c, pl.ds(s * RPS, RPS)])

    @pl.when(s == NS - 1)
    def _flush_tail():
        pltpu.sync_copy(agg_sh.at[pl.ds(RPS * NS, RTAIL)],
                        out_hbm.at[c, pl.ds(RPS * NS, RTAIL)])


_sc_aggregate = pl.kernel(
    _sc_body,
    out_type=jax.ShapeDtypeStruct((NC, N, D), jnp.float32),
    mesh=plsc.VectorSubcoreMesh(
        core_axis_name="c", subcore_axis_name="s",
        num_cores=NC, num_subcores=NS),
    scratch_types=[
        pltpu.VMEM((2, K), jnp.int32),
        pltpu.VMEM((2, K), jnp.int32),
        pltpu.VMEM((EPW + 8,), jnp.float32),
        pltpu.VMEM((K, D), jnp.float32),
        pltpu.VMEM((K, D), jnp.float32),
        pltpu.VMEM_SHARED((N, D), jnp.float32),
        pltpu.SemaphoreType.DMA,
        pltpu.SemaphoreType.DMA,
    ],
    compiler_params=pltpu.CompilerParams(needs_layout_passes=False),
)


def _tc_head_body(agg_ref, batch_ref, wenc_ref, benc_ref, wcls_ref, bcls_ref,
                  out_ref):
    agg = agg_ref[0] + agg_ref[1]
    h = jnp.dot(agg, wenc_ref[...], preferred_element_type=jnp.float32)
    h = jnp.maximum(h + benc_ref[...], 0.0)
    bt = batch_ref[...]
    gids = lax.broadcasted_iota(jnp.int32, (G, N), 0)
    oh = (gids == bt).astype(jnp.float32)
    pooled_sum = jnp.dot(oh, h, preferred_element_type=jnp.float32)
    counts = jnp.sum(oh, axis=1, keepdims=True)
    pooled = pooled_sum / jnp.maximum(counts, 1.0)
    logits = jnp.dot(pooled, wcls_ref[...],
                     preferred_element_type=jnp.float32) + bcls_ref[...]
    out_ref[...] = logits


_tc_head = pl.pallas_call(
    _tc_head_body,
    out_shape=jax.ShapeDtypeStruct((G, C), jnp.float32),
)


def kernel(x, edge_index, batch, edge_weight, W_enc, b_enc, W_cls, b_cls):
    sd = jnp.stack([edge_index[0].reshape(NW, CHUNKS, K),
                    edge_index[1].reshape(NW, CHUNKS, K)], axis=2)
    zeros_nd = jnp.zeros((N, D), jnp.float32)
    agg2 = _sc_aggregate(x, sd, edge_weight, zeros_nd)
    return _tc_head(agg2, batch.reshape(1, N), W_enc, b_enc.reshape(1, D),
                    W_cls, b_cls.reshape(1, C))


# speedup vs baseline: 2.2208x; 1.0004x over previous
import functools

import jax
import jax.numpy as jnp
from jax import lax
from jax.experimental import pallas as pl
from jax.experimental.pallas import tpu as pltpu
from jax.experimental.pallas import tpu_sc as plsc

N = 10000
E = 320000
D = 128
C = 10
G = 64

NC = 2
NS = 16
NW = NC * NS
EPW = E // NW
K = 80
CHUNKS = EPW // K
RPS = 624
RTAIL = N - RPS * NS
LANES = 16


def _sc_body(x_hbm, sd_hbm, w_hbm, zero_hbm, out_hbm,
             sd0_v, sd1_v, w_v, rows0_v, rows1_v, agg_sh, sem0, sem1):
    c = lax.axis_index("c")
    s = lax.axis_index("s")
    wid = c * NS + s

    pltpu.sync_copy(w_hbm.at[pl.ds(wid * EPW, EPW)], w_v.at[pl.ds(8, EPW)])

    pltpu.sync_copy(zero_hbm.at[pl.ds(s * RPS, RPS)],
                    agg_sh.at[pl.ds(s * RPS, RPS)])

    @pl.when(s == NS - 1)
    def _zero_tail():
        pltpu.sync_copy(zero_hbm.at[pl.ds(RPS * NS, RTAIL)],
                        agg_sh.at[pl.ds(RPS * NS, RTAIL)])

    plsc.subcore_barrier()

    def load_sd(g, sd_v):
        pltpu.sync_copy(sd_hbm.at[wid, g], sd_v)

    def gather(sd_v, rows_v, sem):
        return pltpu.make_async_copy(x_hbm.at[sd_v.at[0]], rows_v, sem)

    def scale_and_scatter(g, rows_v, sd_v):
        wbase = g * K + 8

        @plsc.parallel_loop(0, K, unroll=8)
        def _scale(e):
            eidx = jnp.full((LANES,), 0, dtype=jnp.int32) + (e + wbase)
            wspl = plsc.load_gather(w_v, [eidx])
            for j in range(D // LANES):
                sl = rows_v[e, pl.ds(j * LANES, LANES)]
                rows_v[e, pl.ds(j * LANES, LANES)] = sl * wspl

        pltpu.sync_copy(rows_v, agg_sh.at[sd_v.at[1]], add=True)

    load_sd(0, sd0_v)
    gather(sd0_v, rows0_v, sem0).start()

    def pair_body(h, carry):
        g0 = h * 2
        load_sd(g0 + 1, sd1_v)
        gather(sd1_v, rows1_v, sem1).start()
        gather(sd0_v, rows0_v, sem0).wait()
        scale_and_scatter(g0, rows0_v, sd0_v)
        load_sd(g0 + 2, sd0_v)
        gather(sd0_v, rows0_v, sem0).start()
        gather(sd1_v, rows1_v, sem1).wait()
        scale_and_scatter(g0 + 1, rows1_v, sd1_v)
        return carry

    lax.fori_loop(0, CHUNKS // 2, pair_body, 0)
    gather(sd0_v, rows0_v, sem0).wait()
    scale_and_scatter(CHUNKS - 1, rows0_v, sd0_v)
    plsc.subcore_barrier()

    pltpu.sync_copy(agg_sh.at[pl.ds(s * RPS, RPS)],
                    out_hbm.at[c, pl.ds(s * RPS, RPS)])

    @pl.when(s == NS - 1)
    def _flush_tail():
        pltpu.sync_copy(agg_sh.at[pl.ds(RPS * NS, RTAIL)],
                        out_hbm.at[c, pl.ds(RPS * NS, RTAIL)])


_sc_aggregate = pl.kernel(
    _sc_body,
    out_type=jax.ShapeDtypeStruct((NC, N, D), jnp.float32),
    mesh=plsc.VectorSubcoreMesh(
        core_axis_name="c", subcore_axis_name="s",
        num_cores=NC, num_subcores=NS),
    scratch_types=[
        pltpu.VMEM((2, K), jnp.int32),
        pltpu.VMEM((2, K), jnp.int32),
        pltpu.VMEM((EPW + 8,), jnp.float32),
        pltpu.VMEM((K, D), jnp.float32),
        pltpu.VMEM((K, D), jnp.float32),
        pltpu.VMEM_SHARED((N, D), jnp.float32),
        pltpu.SemaphoreType.DMA,
        pltpu.SemaphoreType.DMA,
    ],
    compiler_params=pltpu.CompilerParams(needs_layout_passes=False),
)


def _tc_head_body(agg_ref, batch_ref, wenc_ref, benc_ref, wcls_ref, bcls_ref,
                  out_ref):
    agg = agg_ref[0] + agg_ref[1]
    h = jnp.dot(agg, wenc_ref[...], preferred_element_type=jnp.float32)
    h = jnp.maximum(h + benc_ref[...], 0.0)
    bt = batch_ref[...]
    gids = lax.broadcasted_iota(jnp.int32, (G, N), 0)
    oh = (gids == bt).astype(jnp.float32)
    pooled_sum = jnp.dot(oh, h, preferred_element_type=jnp.float32)
    counts = jnp.sum(oh, axis=1, keepdims=True)
    pooled = pooled_sum / jnp.maximum(counts, 1.0)
    logits = jnp.dot(pooled, wcls_ref[...],
                     preferred_element_type=jnp.float32) + bcls_ref[...]
    out_ref[...] = logits


_tc_head = pl.pallas_call(
    _tc_head_body,
    out_shape=jax.ShapeDtypeStruct((G, C), jnp.float32),
)


def kernel(x, edge_index, batch, edge_weight, W_enc, b_enc, W_cls, b_cls):
    sd = jnp.stack([edge_index[0].reshape(NW, CHUNKS, K),
                    edge_index[1].reshape(NW, CHUNKS, K)], axis=2)
    zeros_nd = jnp.zeros((N, D), jnp.float32)
    agg2 = _sc_aggregate(x, sd, edge_weight, zeros_nd)
    return _tc_head(agg2, batch.reshape(1, N), W_enc, b_enc.reshape(1, D),
                    W_cls, b_cls.reshape(1, C))
